# Initial kernel scaffold; baseline (speedup 1.0000x reference)
import jax
import jax.numpy as jnp
from jax.experimental import pallas as pl


def kernel(x, edge_index):
    raise NotImplementedError("write your pallas kernel here")


# speedup vs baseline: 8.5311x; 8.5311x over previous
import functools

import jax
import jax.numpy as jnp
from jax import lax
from jax.experimental import pallas as pl
from jax.experimental.pallas import tpu as pltpu
from jax.experimental.pallas import tpu_sc as plsc

N_NODES = 10000
D_FEAT = 128
NUM_LAYERS = 3

NCORE = 2
NSUB = 16
NW = NCORE * NSUB
G = 128

N_ACC = 10112
ROWS_PER_SUB = N_ACC // NSUB

_mesh = plsc.VectorSubcoreMesh(core_axis_name="c", subcore_axis_name="s")


def _zero_fill(buf, nrows, width):
    z = jnp.zeros((16,), jnp.float32)

    @pl.loop(0, nrows)
    def _(r):
        @pl.loop(0, width // 16)
        def _(j):
            buf[r, pl.ds(j * 16, 16)] = z


def _copy_rows(src, dst, total_rows):
    off = 0
    while off < total_rows:
        n = min(128, total_rows - off)
        pltpu.sync_copy(src.at[pl.ds(0, n)], dst.at[pl.ds(off, n)])
        off += n


def _make_hist(ngroups):
    pw = ngroups // NW

    @functools.partial(
        pl.kernel,
        out_type=jax.ShapeDtypeStruct((NCORE, N_ACC, 16), jnp.float32),
        mesh=_mesh,
        scratch_types=[
            pltpu.VMEM((2, G), jnp.int32),
            pltpu.VMEM((G, 16), jnp.float32),
            pltpu.VMEM((128, 16), jnp.float32),
            pltpu.VMEM_SHARED((N_ACC, 16), jnp.float32),
        ],
    )
    def hist(col_hbm, out_hbm, idx_v, ones_v, zeros_v, deg_sh):
        c = lax.axis_index("c")
        s = lax.axis_index("s")

        one = jnp.ones((16,), jnp.float32)

        @pl.loop(0, G)
        def _(r):
            ones_v[r, :] = one

        _zero_fill(zeros_v, 128, 16)
        base = s * ROWS_PER_SUB
        off = 0
        while off < ROWS_PER_SUB:
            n = min(128, ROWS_PER_SUB - off)
            pltpu.sync_copy(zeros_v.at[pl.ds(0, n)],
                            deg_sh.at[pl.ds(base + off, n)])
            off += n
        plsc.subcore_barrier()

        g0 = c * (ngroups // NCORE) + s * pw

        @pl.loop(0, pw)
        def _(k):
            pltpu.sync_copy(col_hbm.at[g0 + k], idx_v.at[0])
            pltpu.sync_copy(ones_v, deg_sh.at[idx_v.at[0]], add=True)

        plsc.subcore_barrier()
        pltpu.sync_copy(deg_sh.at[pl.ds(base, ROWS_PER_SUB)],
                        out_hbm.at[c].at[pl.ds(base, ROWS_PER_SUB)])

    return hist


def _make_layer(ngroups):
    pw = ngroups // NW

    @functools.partial(
        pl.kernel,
        out_type=jax.ShapeDtypeStruct((NCORE, N_ACC, D_FEAT), jnp.float32),
        mesh=_mesh,
        scratch_types=[
            pltpu.VMEM((2, G), jnp.int32),
            pltpu.VMEM((2, G), jnp.int32),
            pltpu.VMEM((2, G, D_FEAT), jnp.float32),
            pltpu.VMEM_SHARED((N_ACC, D_FEAT), jnp.float32),
            pltpu.SemaphoreType.DMA,
        ],
    )
    def layer(g_hbm, row_hbm, col_hbm, out_hbm, rix, cix, msgs, acc_sh, sem):
        c = lax.axis_index("c")
        s = lax.axis_index("s")

        _zero_fill(msgs.at[0], G, D_FEAT)
        base = s * ROWS_PER_SUB
        off = 0
        while off < ROWS_PER_SUB:
            n = min(128, ROWS_PER_SUB - off)
            pltpu.sync_copy(msgs.at[0].at[pl.ds(0, n)],
                            acc_sh.at[pl.ds(base + off, n)])
            off += n
        plsc.subcore_barrier()

        g0 = c * (ngroups // NCORE) + s * pw

        @pl.loop(0, pw)
        def _(k):
            pltpu.sync_copy(row_hbm.at[g0 + k], rix.at[0])
            pltpu.sync_copy(col_hbm.at[g0 + k], cix.at[0])
            pltpu.async_copy(g_hbm.at[rix.at[0]], msgs.at[0], sem).wait()
            pltpu.sync_copy(msgs.at[0], acc_sh.at[cix.at[0]], add=True)

        plsc.subcore_barrier()
        pltpu.sync_copy(acc_sh.at[pl.ds(base, ROWS_PER_SUB)],
                        out_hbm.at[c].at[pl.ds(base, ROWS_PER_SUB)])

    return layer


_BLK = 1000
_GRID = N_NODES // _BLK

_deg_spec = pl.BlockSpec((NCORE, _BLK, 16), lambda i: (0, i, 0))
_acc_spec = pl.BlockSpec((NCORE, _BLK, D_FEAT), lambda i: (0, i, 0))
_row_spec = pl.BlockSpec((_BLK, D_FEAT), lambda i: (i, 0))


def _dinv_of(deg_ref):
    deg = deg_ref[0, :, 0:1] + deg_ref[1, :, 0:1] + 1.0
    return lax.rsqrt(deg)


def _prescale_body(deg_ref, x_ref, g_ref):
    g_ref[...] = _dinv_of(deg_ref) * x_ref[...]


_prescale = pl.pallas_call(
    _prescale_body,
    grid=(_GRID,),
    in_specs=[_deg_spec, _row_spec],
    out_specs=_row_spec,
    out_shape=jax.ShapeDtypeStruct((N_NODES, D_FEAT), jnp.float32),
)


def _combine_mid_body(deg_ref, acc_ref, g_ref, sum_ref, gn_ref, sn_ref):
    dinv = _dinv_of(deg_ref)
    h = dinv * (acc_ref[0] + acc_ref[1] + g_ref[...])
    sn_ref[...] = sum_ref[...] + h
    gn_ref[...] = dinv * h


_combine_mid = pl.pallas_call(
    _combine_mid_body,
    grid=(_GRID,),
    in_specs=[_deg_spec, _acc_spec, _row_spec, _row_spec],
    out_specs=[_row_spec, _row_spec],
    out_shape=[jax.ShapeDtypeStruct((N_NODES, D_FEAT), jnp.float32),
               jax.ShapeDtypeStruct((N_NODES, D_FEAT), jnp.float32)],
)


def _combine_last_body(deg_ref, acc_ref, g_ref, sum_ref, out_ref):
    dinv = _dinv_of(deg_ref)
    h = dinv * (acc_ref[0] + acc_ref[1] + g_ref[...])
    out_ref[...] = (sum_ref[...] + h) * (1.0 / (NUM_LAYERS + 1))


_combine_last = pl.pallas_call(
    _combine_last_body,
    grid=(_GRID,),
    in_specs=[_deg_spec, _acc_spec, _row_spec, _row_spec],
    out_specs=_row_spec,
    out_shape=jax.ShapeDtypeStruct((N_NODES, D_FEAT), jnp.float32),
)


def kernel(x, edge_index):
    n_edges = edge_index.shape[1]
    pw = -(-n_edges // (G * NW))
    ngroups = pw * NW
    pad = ngroups * G - n_edges

    row = edge_index[0]
    col = edge_index[1]
    if pad:
        row = jnp.concatenate([row, jnp.zeros((pad,), row.dtype)])
        col = jnp.concatenate([col, jnp.full((pad,), N_NODES, col.dtype)])
    row2d = row.reshape(ngroups, G)
    col2d = col.reshape(ngroups, G)

    hist = _make_hist(ngroups)
    layer = _make_layer(ngroups)

    deg2 = hist(col2d)
    g = _prescale(deg2, x)
    running = x
    for li in range(NUM_LAYERS):
        acc = layer(g, row2d, col2d)
        if li < NUM_LAYERS - 1:
            g, running = _combine_mid(deg2, acc, g, running)
        else:
            out = _combine_last(deg2, acc, g, running)
    return out
